# Initial kernel scaffold; baseline (speedup 1.0000x reference)
import jax
import jax.numpy as jnp
from jax.experimental import pallas as pl


def kernel(actionprobs, bscores, predactions):
    raise NotImplementedError("write your pallas kernel here")


# speedup vs baseline: 43.6548x; 43.6548x over previous
import functools

import jax
import jax.numpy as jnp
from jax import lax
from jax.experimental import pallas as pl
from jax.experimental.pallas import tpu as pltpu
from jax.experimental.pallas import tpu_sc as plsc

B, BEAM, V, T = 64, 8, 32768, 32
ROWS = B * BEAM
NC, NS, L = 2, 16, 16
NW = NC * NS
RPW = ROWS // NW
BPW = B // NW
NGRP = V // (16 * L)
NSLOT = 128
NEG = float("-inf")
BIG = 1 << 30
PN = T + 1
_STAGE = 4


def _sc_body(ap_hbm, bs_hbm, pred_hbm, sc_out, act_out, pn_out,
             row_v, gm_v, cv_v, ci_v, rv_v, ri_v, bs_v, pred_v,
             sc_st, act_st, pn_st, red_f, red_i):
    wid = lax.axis_index("s") * NC + lax.axis_index("c")
    lane = lax.iota(jnp.int32, L)

    def _fold_max_f(ref, base, v):
        a = v
        for off in (8, 4, 2, 1):
            ref[pl.ds(base, L)] = a
            a = jnp.maximum(a, jnp.where(lane < L - off,
                                         ref[pl.ds(base + off, L)], NEG))
        return a[0]

    def _rmax_f(v):
        return _fold_max_f(red_f, 0, v)

    def _rmax_i(v):
        a = v
        for off in (8, 4, 2, 1):
            red_i[pl.ds(0, L)] = a
            a = jnp.maximum(a, jnp.where(lane < L - off,
                                         red_i[pl.ds(off, L)], -BIG))
        return a[0]

    def _rmin_i(v):
        return -_rmax_i(-v)

    pltpu.sync_copy(bs_hbm.at[pl.ds(wid * RPW, RPW)], bs_v)
    pltpu.sync_copy(pred_hbm.at[pl.ds(wid * RPW * T, RPW * T)], pred_v)

    def do_row(rl, _):
        r = wid * RPW + rl
        pltpu.sync_copy(ap_hbm.at[r], row_v)

        def p1(g, m1):
            base = g * (16 * L)
            gv = row_v[pl.ds(base, L)]
            for k in range(1, 16):
                gv = jnp.maximum(gv, row_v[pl.ds(base + k * L, L)])
            gm_v[pl.ds(g * L, L)] = gv
            return jnp.maximum(m1, gv)

        m1 = lax.fori_loop(0, NGRP, p1, jnp.full((L,), NEG))
        for _q in range(7):
            t = _rmax_f(m1)
            m1 = jnp.where(m1 == t, NEG, m1)
        tau = _rmax_f(m1)

        for d in range(8):
            cv_v[pl.ds(d * L, L)] = jnp.full((L,), NEG)
            ci_v[pl.ds(d * L, L)] = jnp.full((L,), BIG)

        def scan_group(g, ns):
            base = g * (16 * L)
            for k in range(16):
                v = row_v[pl.ds(base + k * L, L)]
                vmax = _fold_max_f(red_f, (k + 1) * 2 * L, v)
                hit = vmax >= tau
                m = v >= tau

                @pl.when(hit)
                def _w():
                    n_v = jnp.where(m, v, NEG)
                    n_i = jnp.where(m, base + k * L + lane, BIG)
                    for d in range(8):
                        t_v = cv_v[pl.ds(d * L, L)]
                        t_i = ci_v[pl.ds(d * L, L)]
                        b = (n_v > t_v) | ((n_v == t_v) & (n_i < t_i))
                        cv_v[pl.ds(d * L, L)] = jnp.where(b, n_v, t_v)
                        ci_v[pl.ds(d * L, L)] = jnp.where(b, n_i, t_i)
                        n_v = jnp.where(b, t_v, n_v)
                        n_i = jnp.where(b, t_i, n_i)

                ns = ns + jnp.where(hit, 1, 0)
            return ns

        def p2(g, ns):
            gv = gm_v[pl.ds(g * L, L)]
            gmax = _fold_max_f(gm_v, g * L, gv)
            return lax.cond(gmax >= tau,
                            functools.partial(scan_group, g),
                            lambda c: c, ns)

        if _STAGE < 2:
            rv_v[pl.ds(rl * L, L)] = jnp.full((L,), tau)
            ri_v[pl.ds(rl * L, L)] = jnp.zeros((L,), jnp.int32)
            return _
        ns = lax.fori_loop(0, NGRP, p2, jnp.int32(0))
        nsc = jnp.minimum(ns, NSLOT)

        if _STAGE < 3:
            rv_v[pl.ds(rl * L, L)] = jnp.full((L,), tau) + nsc
            ri_v[pl.ds(rl * L, L)] = jnp.zeros((L,), jnp.int32)
            return _
        rvacc = jnp.full((L,), NEG)
        riacc = jnp.zeros((L,), jnp.int32)
        lastv = jnp.float32(jnp.inf)
        lasti = jnp.int32(-1)
        stks = [(cv_v[pl.ds(d * L, L)], ci_v[pl.ds(d * L, L)])
                for d in range(8)]
        for p in range(8):
            am = jnp.full((L,), NEG)
            ai = jnp.full((L,), BIG)
            for v0, iv0 in stks:
                ok = (v0 < lastv) | ((v0 == lastv) & (iv0 > lasti))
                v = jnp.where(ok, v0, NEG)
                iv = jnp.where(ok, iv0, BIG)
                better = (v > am) | ((v == am) & (iv < ai))
                am = jnp.where(better, v, am)
                ai = jnp.where(better, iv, ai)
            m = _rmax_f(am)
            bi = _rmin_i(jnp.where(am == m, ai, BIG))
            rvacc = jnp.where(lane == p, m, rvacc)
            riacc = jnp.where(lane == p, bi, riacc)
            lastv, lasti = m, bi
        rv_v[pl.ds(rl * L, L)] = rvacc
        ri_v[pl.ds(rl * L, L)] = riacc
        return _

    lax.fori_loop(0, RPW, do_row, jnp.int32(0))

    if _STAGE < 4:
        sc_st[pl.ds(0, RPW)] = rv_v[pl.ds(0, RPW)]
        act_st[pl.ds(0, RPW)] = ri_v[pl.ds(0, RPW)]
        for _z in range(BPW * BEAM * PN // L + 1):
            pn_st[pl.ds(_z * L, L)] = jnp.zeros((L,), jnp.int32)
        pltpu.sync_copy(sc_st, sc_out.at[pl.ds(wid * RPW, RPW)])
        pltpu.sync_copy(act_st, act_out.at[pl.ds(wid * RPW, RPW)])
        pltpu.sync_copy(pn_st.at[pl.ds(0, BPW * BEAM * PN)],
                        pn_out.at[pl.ds(wid * BPW * BEAM * PN,
                                        BPW * BEAM * PN)])
        return
    bsall = bs_v[pl.ds(0, RPW)]
    scacc = jnp.zeros((L,), jnp.float32)
    actacc = jnp.zeros((L,), jnp.int32)
    for bl in range(BPW):
        svs, ivs, fvs = [], [], []
        for beam in range(BEAM):
            rloc = bl * BEAM + beam
            sv = rv_v[pl.ds(rloc * L, L)]
            sv = jnp.where(lane < BEAM, sv + bsall[rloc], NEG)
            svs.append(sv)
            ivs.append(ri_v[pl.ds(rloc * L, L)])
            fvs.append(jnp.where(lane < BEAM, beam * BEAM + lane, BIG))
        for i in range(BEAM):
            am, ai, av = svs[0], fvs[0], ivs[0]
            for beam in range(1, BEAM):
                better = ((svs[beam] > am)
                          | ((svs[beam] == am) & (fvs[beam] < ai)))
                am = jnp.where(better, svs[beam], am)
                ai = jnp.where(better, fvs[beam], ai)
                av = jnp.where(better, ivs[beam], av)
            m = _rmax_f(am)
            f = _rmin_i(jnp.where(am == m, ai, BIG))
            aid = _rmax_i(jnp.where((am == m) & (ai == f), av, -1))
            sid = f // BEAM
            for beam in range(BEAM):
                hit = (svs[beam] == m) & (fvs[beam] == f)
                svs[beam] = jnp.where(hit, NEG, svs[beam])
            scacc = jnp.where(lane == bl * BEAM + i, m, scacc)
            actacc = jnp.where(lane == bl * BEAM + i, aid, actacc)
            pb = (bl * BEAM + sid) * T
            o = (bl * BEAM + i) * PN
            pn_st[pl.ds(o, L)] = pred_v[pl.ds(pb, L)]
            pn_st[pl.ds(o + L, L)] = pred_v[pl.ds(pb + L, L)]
            pn_st[pl.ds(o + T, L)] = jnp.zeros((L,), jnp.int32) + aid
    sc_st[pl.ds(0, RPW)] = scacc
    act_st[pl.ds(0, RPW)] = actacc

    pltpu.sync_copy(sc_st, sc_out.at[pl.ds(wid * RPW, RPW)])
    pltpu.sync_copy(act_st, act_out.at[pl.ds(wid * RPW, RPW)])
    pltpu.sync_copy(pn_st.at[pl.ds(0, BPW * BEAM * PN)],
                    pn_out.at[pl.ds(wid * BPW * BEAM * PN, BPW * BEAM * PN)])


@jax.jit
def kernel(actionprobs, bscores, predactions):
    ap = actionprobs.reshape(ROWS, V)
    bs = bscores.reshape(ROWS)
    pred = predactions.reshape(ROWS * T)

    call = pl.kernel(
        _sc_body,
        out_type=(
            jax.ShapeDtypeStruct((ROWS,), jnp.float32),
            jax.ShapeDtypeStruct((ROWS,), jnp.int32),
            jax.ShapeDtypeStruct((B * BEAM * PN,), jnp.int32),
        ),
        mesh=plsc.VectorSubcoreMesh(core_axis_name="c", subcore_axis_name="s"),
        scratch_types=[
            pltpu.VMEM((V,), jnp.float32),
            pltpu.VMEM((NGRP * L + L,), jnp.float32),
            pltpu.VMEM((NSLOT * L,), jnp.float32),
            pltpu.VMEM((NSLOT * L,), jnp.int32),
            pltpu.VMEM((RPW * L,), jnp.float32),
            pltpu.VMEM((RPW * L,), jnp.int32),
            pltpu.VMEM((RPW,), jnp.float32),
            pltpu.VMEM((RPW * T,), jnp.int32),
            pltpu.VMEM((RPW,), jnp.float32),
            pltpu.VMEM((RPW,), jnp.int32),
            pltpu.VMEM((BPW * BEAM * PN + L,), jnp.int32),
            pltpu.VMEM((17 * 2 * L,), jnp.float32),
            pltpu.VMEM((2 * L,), jnp.int32),
        ],
    )
    scores, actionids, pred_new = call(ap, bs, pred)
    return (scores.reshape(B, BEAM), actionids.reshape(B, BEAM),
            pred_new.reshape(B, BEAM, PN))


# speedup vs baseline: 43.6616x; 1.0002x over previous
import functools

import jax
import jax.numpy as jnp
from jax import lax
from jax.experimental import pallas as pl
from jax.experimental.pallas import tpu as pltpu
from jax.experimental.pallas import tpu_sc as plsc

B, BEAM, V, T = 64, 8, 32768, 32
ROWS = B * BEAM
NC, NS, L = 2, 16, 16
NW = NC * NS
RPW = ROWS // NW
BPW = B // NW
NGRP = V // (16 * L)
NEG = float("-inf")
BIG = 1 << 30
PN = T + 1


def _sc_body(ap_hbm, bs_hbm, pred_hbm, sc_out, act_out, pn_out,
             row_v, gm_v, cv_v, ci_v, rv_v, ri_v, bs_v, pred_v,
             sc_st, act_st, pn_st, red_f, red_i):
    wid = lax.axis_index("s") * NC + lax.axis_index("c")
    lane = lax.iota(jnp.int32, L)

    def _fold_max_f(ref, base, v):
        a = v
        for off in (8, 4, 2, 1):
            ref[pl.ds(base, L)] = a
            a = jnp.maximum(a, jnp.where(lane < L - off,
                                         ref[pl.ds(base + off, L)], NEG))
        return a[0]

    def _rmax_f(v):
        return _fold_max_f(red_f, 0, v)

    def _rmax_i(v):
        a = v
        for off in (8, 4, 2, 1):
            red_i[pl.ds(0, L)] = a
            a = jnp.maximum(a, jnp.where(lane < L - off,
                                         red_i[pl.ds(off, L)], -BIG))
        return a[0]

    def _rmin_i(v):
        return -_rmax_i(-v)

    pltpu.sync_copy(bs_hbm.at[pl.ds(wid * RPW, RPW)], bs_v)
    pltpu.sync_copy(pred_hbm.at[pl.ds(wid * RPW * T, RPW * T)], pred_v)

    def do_row(rl, _):
        r = wid * RPW + rl
        pltpu.sync_copy(ap_hbm.at[r], row_v)

        def p1(g, m1):
            base = g * (16 * L)
            gv = row_v[pl.ds(base, L)]
            for k in range(1, 16):
                gv = jnp.maximum(gv, row_v[pl.ds(base + k * L, L)])
            gm_v[pl.ds(g * L, L)] = gv
            return jnp.maximum(m1, gv)

        m1 = lax.fori_loop(0, NGRP, p1, jnp.full((L,), NEG))
        for _q in range(7):
            t = _rmax_f(m1)
            m1 = jnp.where(m1 == t, NEG, m1)
        tau = _rmax_f(m1)

        for d in range(8):
            cv_v[pl.ds(d * L, L)] = jnp.full((L,), NEG)
            ci_v[pl.ds(d * L, L)] = jnp.full((L,), BIG)

        def scan_group(g, ns):
            base = g * (16 * L)
            for k in range(16):
                v = row_v[pl.ds(base + k * L, L)]
                vmax = _fold_max_f(red_f, (k + 1) * 2 * L, v)
                hit = vmax >= tau
                m = v >= tau

                @pl.when(hit)
                def _w():
                    n_v = jnp.where(m, v, NEG)
                    n_i = jnp.where(m, base + k * L + lane, BIG)
                    for d in range(8):
                        t_v = cv_v[pl.ds(d * L, L)]
                        t_i = ci_v[pl.ds(d * L, L)]
                        b = (n_v > t_v) | ((n_v == t_v) & (n_i < t_i))
                        cv_v[pl.ds(d * L, L)] = jnp.where(b, n_v, t_v)
                        ci_v[pl.ds(d * L, L)] = jnp.where(b, n_i, t_i)
                        n_v = jnp.where(b, t_v, n_v)
                        n_i = jnp.where(b, t_i, n_i)

            return ns

        def p2(g, ns):
            gv = gm_v[pl.ds(g * L, L)]
            gmax = _fold_max_f(gm_v, g * L, gv)
            return lax.cond(gmax >= tau,
                            functools.partial(scan_group, g),
                            lambda c: c, ns)

        lax.fori_loop(0, NGRP, p2, jnp.int32(0))

        rvacc = jnp.full((L,), NEG)
        riacc = jnp.zeros((L,), jnp.int32)
        lastv = jnp.float32(jnp.inf)
        lasti = jnp.int32(-1)
        stks = [(cv_v[pl.ds(d * L, L)], ci_v[pl.ds(d * L, L)])
                for d in range(8)]
        for p in range(8):
            am = jnp.full((L,), NEG)
            ai = jnp.full((L,), BIG)
            for v0, iv0 in stks:
                ok = (v0 < lastv) | ((v0 == lastv) & (iv0 > lasti))
                v = jnp.where(ok, v0, NEG)
                iv = jnp.where(ok, iv0, BIG)
                better = (v > am) | ((v == am) & (iv < ai))
                am = jnp.where(better, v, am)
                ai = jnp.where(better, iv, ai)
            m = _rmax_f(am)
            bi = _rmin_i(jnp.where(am == m, ai, BIG))
            rvacc = jnp.where(lane == p, m, rvacc)
            riacc = jnp.where(lane == p, bi, riacc)
            lastv, lasti = m, bi
        rv_v[pl.ds(rl * L, L)] = rvacc
        ri_v[pl.ds(rl * L, L)] = riacc
        return _

    lax.fori_loop(0, RPW, do_row, jnp.int32(0))

    bsall = bs_v[pl.ds(0, RPW)]
    scacc = jnp.zeros((L,), jnp.float32)
    actacc = jnp.zeros((L,), jnp.int32)
    for bl in range(BPW):
        svs, ivs, fvs = [], [], []
        for beam in range(BEAM):
            rloc = bl * BEAM + beam
            sv = rv_v[pl.ds(rloc * L, L)]
            sv = jnp.where(lane < BEAM, sv + bsall[rloc], NEG)
            svs.append(sv)
            ivs.append(ri_v[pl.ds(rloc * L, L)])
            fvs.append(jnp.where(lane < BEAM, beam * BEAM + lane, BIG))
        for i in range(BEAM):
            am, ai, av = svs[0], fvs[0], ivs[0]
            for beam in range(1, BEAM):
                better = ((svs[beam] > am)
                          | ((svs[beam] == am) & (fvs[beam] < ai)))
                am = jnp.where(better, svs[beam], am)
                ai = jnp.where(better, fvs[beam], ai)
                av = jnp.where(better, ivs[beam], av)
            m = _rmax_f(am)
            f = _rmin_i(jnp.where(am == m, ai, BIG))
            aid = _rmax_i(jnp.where((am == m) & (ai == f), av, -1))
            sid = f // BEAM
            for beam in range(BEAM):
                hit = (svs[beam] == m) & (fvs[beam] == f)
                svs[beam] = jnp.where(hit, NEG, svs[beam])
            scacc = jnp.where(lane == bl * BEAM + i, m, scacc)
            actacc = jnp.where(lane == bl * BEAM + i, aid, actacc)
            pb = (bl * BEAM + sid) * T
            o = (bl * BEAM + i) * PN
            pn_st[pl.ds(o, L)] = pred_v[pl.ds(pb, L)]
            pn_st[pl.ds(o + L, L)] = pred_v[pl.ds(pb + L, L)]
            pn_st[pl.ds(o + T, L)] = jnp.zeros((L,), jnp.int32) + aid
    sc_st[pl.ds(0, RPW)] = scacc
    act_st[pl.ds(0, RPW)] = actacc

    pltpu.sync_copy(sc_st, sc_out.at[pl.ds(wid * RPW, RPW)])
    pltpu.sync_copy(act_st, act_out.at[pl.ds(wid * RPW, RPW)])
    pltpu.sync_copy(pn_st.at[pl.ds(0, BPW * BEAM * PN)],
                    pn_out.at[pl.ds(wid * BPW * BEAM * PN, BPW * BEAM * PN)])


@jax.jit
def kernel(actionprobs, bscores, predactions):
    ap = actionprobs.reshape(ROWS, V)
    bs = bscores.reshape(ROWS)
    pred = predactions.reshape(ROWS * T)

    call = pl.kernel(
        _sc_body,
        out_type=(
            jax.ShapeDtypeStruct((ROWS,), jnp.float32),
            jax.ShapeDtypeStruct((ROWS,), jnp.int32),
            jax.ShapeDtypeStruct((B * BEAM * PN,), jnp.int32),
        ),
        mesh=plsc.VectorSubcoreMesh(core_axis_name="c", subcore_axis_name="s"),
        scratch_types=[
            pltpu.VMEM((V,), jnp.float32),
            pltpu.VMEM((NGRP * L + L,), jnp.float32),
            pltpu.VMEM((8 * L,), jnp.float32),
            pltpu.VMEM((8 * L,), jnp.int32),
            pltpu.VMEM((RPW * L,), jnp.float32),
            pltpu.VMEM((RPW * L,), jnp.int32),
            pltpu.VMEM((RPW,), jnp.float32),
            pltpu.VMEM((RPW * T,), jnp.int32),
            pltpu.VMEM((RPW,), jnp.float32),
            pltpu.VMEM((RPW,), jnp.int32),
            pltpu.VMEM((BPW * BEAM * PN + L,), jnp.int32),
            pltpu.VMEM((17 * 2 * L,), jnp.float32),
            pltpu.VMEM((2 * L,), jnp.int32),
        ],
    )
    scores, actionids, pred_new = call(ap, bs, pred)
    return (scores.reshape(B, BEAM), actionids.reshape(B, BEAM),
            pred_new.reshape(B, BEAM, PN))
